# Initial kernel scaffold; baseline (speedup 1.0000x reference)
import jax
import jax.numpy as jnp
from jax import lax
from jax.experimental import pallas as pl
from jax.experimental.pallas import tpu as pltpu
from jax.experimental.pallas import tpu_sc as plsc

N = 10000
E = 160000
FEAT = 256
HID = 128
HEADS = 4
NG = 3
NP = 10112
TSL = NP // 16
CH = 2528
RT = CH // 16
EB = 256
NB_A = E // EB
SB = 2000
NVB = SB // 16
NBLK = (E // 16) // SB
K = 96
KV = K // 16


def _dense_body(x_ref, w_ref, a_ref, proj_ref, s_ref):
    p = jnp.dot(x_ref[...], w_ref[...], preferred_element_type=jnp.float32)
    proj_ref[...] = p
    s_ref[...] = jnp.dot(p, a_ref[...], preferred_element_type=jnp.float32)


def _rden_body(p_ref, o_ref):
    o_ref[...] = 1.0 / (p_ref[0] + p_ref[1] + 1e-9)


def _pass_a(sl_h, sr_h, ei_h, pw_h, zden_h, wnum_h, denp_h,
            slg_v, srg_v, sv, dv, pv, exst, wnst, den_sh, sem):
    s = lax.axis_index("s")
    c = lax.axis_index("c")
    wid = s * 2 + c
    nb = jnp.where(wid < (NB_A % 32), NB_A // 32 + 1, NB_A // 32)

    def graph_body(g, _):
        pltpu.sync_copy(sl_h.at[g], slg_v)
        pltpu.sync_copy(sr_h.at[g], srg_v)
        pltpu.sync_copy(zden_h, den_sh.at[:, pl.ds(s * TSL, TSL)])
        plsc.subcore_barrier()

        def batch_body(i, _):
            b = wid + i * 32
            eb = b * EB
            pltpu.sync_copy(ei_h.at[g, 0, pl.ds(eb, EB)], sv)
            pltpu.sync_copy(ei_h.at[g, 1, pl.ds(eb, EB)], dv)
            pltpu.sync_copy(pw_h.at[g, pl.ds(eb, EB)], pv)

            def vec_body(v, _):
                srci = sv[pl.ds(v * 16, 16)]
                dsti = dv[pl.ds(v * 16, 16)]
                pw = pv[pl.ds(v * 16, 16)]
                for h in range(HEADS):
                    hh = jnp.full((16,), h, jnp.int32)
                    slv = plsc.load_gather(slg_v, [srci, hh])
                    srv = plsc.load_gather(srg_v, [dsti, hh])
                    e = pw * slv + srv
                    e = jnp.where(e > 0, e, e * 0.01)
                    ex = jnp.exp(e)
                    exst[h, pl.ds(v * 16, 16)] = ex
                    wnst[h, pl.ds(v * 16, 16)] = ex * pw
                return 0

            lax.fori_loop(0, EB // 16, vec_body, 0)
            for h in range(HEADS):
                pltpu.sync_copy(exst.at[h], den_sh.at[h].at[dv], add=True)
            pltpu.sync_copy(wnst, wnum_h.at[g, :, pl.ds(eb, EB)])
            return 0

        lax.fori_loop(0, nb, batch_body, 0)
        plsc.subcore_barrier()
        pltpu.sync_copy(den_sh.at[:, pl.ds(s * TSL, TSL)],
                        denp_h.at[c, g, :, pl.ds(s * TSL, TSL)])
        plsc.subcore_barrier()
        return 0

    lax.fori_loop(0, NG, graph_body, 0)


def _pass_b(proj_h, ei_h, wnum_h, rden_h, zacc_h, outg_h,
            den_v, sblk, dblk, csrc_b, cdst_b, cid_b, csrcK, cdstK, cidK,
            whb, wk, rows_v, stage_v, acc_sh, sem):
    s = lax.axis_index("s")
    c = lax.axis_index("c")
    lanes = lax.iota(jnp.int32, 16)

    def process(g, lo, cnt_eff):
        for vb in range(KV):
            valid = (vb * 16 + lanes) < cnt_eff
            csrcK[pl.ds(vb * 16, 16)] = jnp.where(valid, csrc_b[pl.ds(vb * 16, 16)], 0)
            cdstK[pl.ds(vb * 16, 16)] = jnp.where(valid, cdst_b[pl.ds(vb * 16, 16)], 0)
            cidK[pl.ds(vb * 16, 16)] = jnp.where(valid, cid_b[pl.ds(vb * 16, 16)], 0)
        pltpu.async_copy(proj_h.at[csrcK], rows_v, sem).wait()
        for h in range(HEADS):
            pltpu.async_copy(wnum_h.at[g, h].at[cidK], whb.at[h], sem).wait()
        for h in range(HEADS):
            hh = jnp.full((16,), h, jnp.int32)
            for vb in range(KV):
                dvl = cdstK[pl.ds(vb * 16, 16)]
                rd = plsc.load_gather(den_v, [hh, dvl + lo])
                wn = whb[h, pl.ds(vb * 16, 16)]
                valid = (vb * 16 + lanes) < cnt_eff
                wk[h, pl.ds(vb * 16, 16)] = jnp.where(valid, wn * rd, 0.0)

        def edge_body(k, _):
            rows = [rows_v[k, pl.ds(j * 16, 16)] for j in range(HID // 16)]
            for h in range(HEADS):
                wv = jnp.full((16,), wk[h, k], jnp.float32)
                for j in range(HID // 16):
                    stage_v[k, pl.ds(h * HID + j * 16, 16)] = rows[j] * wv
            return 0

        lax.fori_loop(0, K, edge_body, 0)
        pltpu.sync_copy(stage_v, acc_sh.at[cdstK], add=True)

    def graph_body(g, _):
        pltpu.sync_copy(rden_h.at[g], den_v)

        def chunk_body(ch, _):
            lo = (ch * 2 + c) * CH
            pltpu.sync_copy(zacc_h, acc_sh.at[pl.ds(s * RT, RT)])
            plsc.subcore_barrier()

            def blk_body(blk, cnt):
                ebase = s * (E // 16) + blk * SB
                pltpu.sync_copy(ei_h.at[g, 0, pl.ds(ebase, SB)], sblk)
                pltpu.sync_copy(ei_h.at[g, 1, pl.ds(ebase, SB)], dblk)

                def vec_body(v, cnt):
                    srci = sblk[pl.ds(v * 16, 16)]
                    dsti = dblk[pl.ds(v * 16, 16)]
                    m = (dsti >= lo) & (dsti < lo + CH)
                    eid = ebase + v * 16 + lanes
                    plsc.store_compressed(csrc_b.at[pl.ds(cnt, 16)], srci, mask=m)
                    plsc.store_compressed(cdst_b.at[pl.ds(cnt, 16)], dsti - lo, mask=m)
                    plsc.store_compressed(cid_b.at[pl.ds(cnt, 16)], eid, mask=m)
                    cnt = cnt + jnp.sum(m.astype(jnp.int32))

                    @pl.when(cnt >= K)
                    def _():
                        process(g, lo, jnp.int32(K))
                        csrc_b[pl.ds(0, 16)] = csrc_b[pl.ds(K, 16)]
                        cdst_b[pl.ds(0, 16)] = cdst_b[pl.ds(K, 16)]
                        cid_b[pl.ds(0, 16)] = cid_b[pl.ds(K, 16)]

                    return jnp.where(cnt >= K, cnt - K, cnt)

                return lax.fori_loop(0, NVB, vec_body, cnt)

            cnt = lax.fori_loop(0, NBLK, blk_body, jnp.int32(0))

            @pl.when(cnt > 0)
            def _():
                process(g, lo, cnt)

            plsc.subcore_barrier()
            pltpu.sync_copy(acc_sh.at[pl.ds(s * RT, RT)],
                            outg_h.at[g, pl.ds(lo + s * RT, RT)])
            plsc.subcore_barrier()
            return 0

        lax.fori_loop(0, 2, chunk_body, 0)
        return 0

    lax.fori_loop(0, NG, graph_body, 0)


def kernel(x, fg_edge_index, fg_pre_w, bg_edge_index, bg_pre_w,
           gg_edge_index, gg_pre_w, W_proj, att_l, att_r):
    f32 = jnp.float32
    i32 = jnp.int32

    A = jnp.concatenate([att_l.reshape(NG * HEADS, HID).T,
                         att_r.reshape(NG * HEADS, HID).T], axis=1)
    rb = 2000
    proj, S = pl.pallas_call(
        _dense_body,
        grid=(N // rb,),
        in_specs=[pl.BlockSpec((rb, FEAT), lambda i: (i, 0)),
                  pl.BlockSpec((FEAT, HID), lambda i: (0, 0)),
                  pl.BlockSpec((HID, 2 * NG * HEADS), lambda i: (0, 0))],
        out_specs=[pl.BlockSpec((rb, HID), lambda i: (i, 0)),
                   pl.BlockSpec((rb, 2 * NG * HEADS), lambda i: (i, 0))],
        out_shape=[jax.ShapeDtypeStruct((N, HID), f32),
                   jax.ShapeDtypeStruct((N, 2 * NG * HEADS), f32)],
    )(x, W_proj, A)

    SLm = jnp.transpose(S[:, :NG * HEADS].reshape(N, NG, HEADS), (1, 0, 2))
    SRm = jnp.transpose(S[:, NG * HEADS:].reshape(N, NG, HEADS), (1, 0, 2))
    SL = jnp.pad(SLm, ((0, 0), (0, NP - N), (0, 0)))
    SR = jnp.pad(SRm, ((0, 0), (0, NP - N), (0, 0)))
    ei = jnp.stack([fg_edge_index, bg_edge_index, gg_edge_index]).astype(i32)
    prew = jnp.stack([fg_pre_w, bg_pre_w, gg_pre_w]).astype(f32)
    zden = jnp.zeros((HEADS, TSL), f32)
    zacc = jnp.zeros((RT, HEADS * HID), f32)

    mesh = plsc.VectorSubcoreMesh(core_axis_name="c", subcore_axis_name="s")

    wnum, denp = pl.kernel(
        _pass_a,
        out_type=(jax.ShapeDtypeStruct((NG, HEADS, E), f32),
                  jax.ShapeDtypeStruct((2, NG, HEADS, NP), f32)),
        mesh=mesh,
        scratch_types=[
            pltpu.VMEM((NP, HEADS), f32),
            pltpu.VMEM((NP, HEADS), f32),
            pltpu.VMEM((EB,), i32),
            pltpu.VMEM((EB,), i32),
            pltpu.VMEM((EB,), f32),
            pltpu.VMEM((HEADS, EB), f32),
            pltpu.VMEM((HEADS, EB), f32),
            pltpu.VMEM_SHARED((HEADS, NP), f32),
            pltpu.SemaphoreType.DMA,
        ],
    )(SL, SR, ei, prew, zden)

    rden = pl.pallas_call(
        _rden_body,
        out_shape=jax.ShapeDtypeStruct((NG, HEADS, NP), f32),
    )(denp)

    outg = pl.kernel(
        _pass_b,
        out_type=jax.ShapeDtypeStruct((NG, NP, HEADS * HID), f32),
        mesh=mesh,
        scratch_types=[
            pltpu.VMEM((HEADS, NP), f32),
            pltpu.VMEM((SB,), i32),
            pltpu.VMEM((SB,), i32),
            pltpu.VMEM((K + 16,), i32),
            pltpu.VMEM((K + 16,), i32),
            pltpu.VMEM((K + 16,), i32),
            pltpu.VMEM((K,), i32),
            pltpu.VMEM((K,), i32),
            pltpu.VMEM((K,), i32),
            pltpu.VMEM((HEADS, K), f32),
            pltpu.VMEM((HEADS, K), f32),
            pltpu.VMEM((K, HID), f32),
            pltpu.VMEM((K, HEADS * HID), f32),
            pltpu.VMEM_SHARED((CH, HEADS * HID), f32),
            pltpu.SemaphoreType.DMA,
        ],
    )(proj, ei, wnum, rden, zacc)

    out = jnp.concatenate(
        [proj, outg[0, :N], outg[1, :N], outg[2, :N]], axis=-1)
    return out[None]


# speedup vs baseline: 14.7601x; 14.7601x over previous
import jax
import jax.numpy as jnp
from jax import lax
from jax.experimental import pallas as pl
from jax.experimental.pallas import tpu as pltpu
from jax.experimental.pallas import tpu_sc as plsc

N = 10000
E = 160000
FEAT = 256
HID = 128
HEADS = 4
NG = 3
NW = 32
NP = 10240
TSL = NP // 16
CR = 160
SW = NP // (NW * CR)
EB = 800
NB_A = E // EB
SB = 8000
NVB = SB // 16
NBLK = E // SB
K = 192
KV = K // 16
DH = HEADS * HID


def _dense_body(x_ref, w_ref, a_ref, proj_ref, s_ref):
    p = jnp.dot(x_ref[...], w_ref[...], preferred_element_type=jnp.float32)
    proj_ref[...] = p
    s_ref[...] = jnp.dot(p, a_ref[...], preferred_element_type=jnp.float32)


def _rden_body(p_ref, o_ref):
    o_ref[...] = 1.0 / (p_ref[0] + p_ref[1] + 1e-9)


def _pass_a(sl_h, sr_h, srcs_h, dsts_h, pw_h, zden_h,
            wn0_h, wn1_h, wn2_h, wn3_h, denp_h,
            slg_v, srg_v, sv, dv, pv, exst, wnst, hidx, den_sh, sem):
    s = lax.axis_index("s")
    c = lax.axis_index("c")
    wid = s * 2 + c
    nb = jnp.where(wid < (NB_A % NW), NB_A // NW + 1, NB_A // NW)
    wn_h = [wn0_h, wn1_h, wn2_h, wn3_h]

    def graph_body(g, _):
        pltpu.sync_copy(sl_h.at[pl.ds(g * NP * HEADS, NP * HEADS)], slg_v)
        pltpu.sync_copy(sr_h.at[pl.ds(g * NP * HEADS, NP * HEADS)], srg_v)
        pltpu.sync_copy(zden_h, den_sh.at[pl.ds(s * HEADS * TSL, HEADS * TSL)])
        plsc.subcore_barrier()

        def batch_body(i, _):
            b = wid + i * NW
            eb = b * EB
            d1 = pltpu.async_copy(srcs_h.at[pl.ds(g * E + eb, EB)], sv, sem)
            d2 = pltpu.async_copy(dsts_h.at[pl.ds(g * E + eb, EB)], dv, sem)
            d3 = pltpu.async_copy(pw_h.at[pl.ds(g * E + eb, EB)], pv, sem)
            d1.wait()
            d2.wait()
            d3.wait()

            def vec_body(v, _):
                srci = sv[pl.ds(v * 16, 16)]
                dsti = dv[pl.ds(v * 16, 16)]
                pw = pv[pl.ds(v * 16, 16)]
                src4 = srci * HEADS
                dst4 = dsti * HEADS
                for h in range(HEADS):
                    slv = plsc.load_gather(slg_v, [src4 + h])
                    srv = plsc.load_gather(srg_v, [dst4 + h])
                    e = pw * slv + srv
                    e = jnp.where(e > 0, e, e * 0.01)
                    ex = jnp.exp(e)
                    exst[h][pl.ds(v * 16, 16)] = ex
                    wnst[h][pl.ds(v * 16, 16)] = ex * pw
                    hidx[h][pl.ds(v * 16, 16)] = dsti + h * NP
                return 0

            lax.fori_loop(0, EB // 16, vec_body, 0)
            for h in range(HEADS):
                pltpu.sync_copy(exst[h], den_sh.at[hidx[h]], add=True)
                pltpu.sync_copy(wnst[h], wn_h[h].at[pl.ds(g * E + eb, EB)])
            return 0

        lax.fori_loop(0, nb, batch_body, 0)
        plsc.subcore_barrier()
        pltpu.sync_copy(
            den_sh.at[pl.ds(s * HEADS * TSL, HEADS * TSL)],
            denp_h.at[pl.ds((c * NG + g) * HEADS * NP + s * HEADS * TSL,
                            HEADS * TSL)])
        plsc.subcore_barrier()
        return 0

    lax.fori_loop(0, NG, graph_body, 0)


def _pass_b(proj_h, srcs_h, dsts_h, wn0_h, wn1_h, wn2_h, wn3_h, rden_h,
            zacc_h, outg_h,
            dblk, cid_b, cdst_b, cidK, cdstK, srcv, whb, wk, den_v,
            rows_v, acc_v, sem, sem2):
    s = lax.axis_index("s")
    c = lax.axis_index("c")
    wid = s * 2 + c
    lanes = lax.iota(jnp.int32, 16)
    wn_h = [wn0_h, wn1_h, wn2_h, wn3_h]

    def process(g, cnt_eff):
        for vb in range(KV):
            valid = (vb * 16 + lanes) < cnt_eff
            cidK[pl.ds(vb * 16, 16)] = jnp.where(
                valid, cid_b[pl.ds(vb * 16, 16)], g * E)
            cdstK[pl.ds(vb * 16, 16)] = jnp.where(
                valid, cdst_b[pl.ds(vb * 16, 16)], 0)
        d_src = pltpu.async_copy(srcs_h.at[cidK], srcv, sem2)
        d_wn = [pltpu.async_copy(wn_h[h].at[cidK], whb[h], sem)
                for h in range(HEADS)]
        d_src.wait()
        d_rows = pltpu.async_copy(proj_h.at[srcv], rows_v, sem2)
        for d in d_wn:
            d.wait()
        for h in range(HEADS):
            for vb in range(KV):
                dvl = cdstK[pl.ds(vb * 16, 16)]
                rd = plsc.load_gather(den_v, [dvl + h * CR])
                wn = whb[h][pl.ds(vb * 16, 16)]
                valid = (vb * 16 + lanes) < cnt_eff
                wk[h][pl.ds(vb * 16, 16)] = jnp.where(valid, wn * rd, 0.0)
        d_rows.wait()

        def group_body(kb, _):
            cvec = cdstK[pl.ds(kb * 16, 16)]
            wvecs = [wk[h][pl.ds(kb * 16, 16)] for h in range(HEADS)]
            for kk in range(16):
                k = kb * 16 + kk
                rows = [rows_v[k, pl.ds(j * 16, 16)] for j in range(HID // 16)]
                base = cvec[kk] * DH
                for h in range(HEADS):
                    wsc = wvecs[h][kk]
                    for j in range(HID // 16):
                        plsc.addupdate(
                            acc_v.at[pl.ds(base + (h * HID + j * 16), 16)],
                            rows[j] * wsc)
            return 0

        lax.fori_loop(0, KV, group_body, 0)

    def graph_body(g, _):
        def sweep_body(w, _):
            rowbase = (w * NW + wid) * CR
            for h in range(HEADS):
                pltpu.sync_copy(
                    rden_h.at[pl.ds((g * HEADS + h) * NP + rowbase, CR)],
                    den_v.at[pl.ds(h * CR, CR)])
            pltpu.sync_copy(zacc_h, acc_v)

            def blk_body(blk, cnt):
                ebase = blk * SB
                pltpu.sync_copy(dsts_h.at[pl.ds(g * E + ebase, SB)], dblk)

                def vec4_body(v4, cnt):
                    for u in range(4):
                        v16 = v4 * 64 + u * 16
                        dsti = dblk[pl.ds(v16, 16)]
                        m = (dsti >= rowbase) & (dsti < rowbase + CR)
                        eid = (g * E + ebase + v16) + lanes
                        plsc.store_compressed(cid_b.at[pl.ds(cnt, 16)],
                                              eid, mask=m)
                        plsc.store_compressed(cdst_b.at[pl.ds(cnt, 16)],
                                              dsti - rowbase, mask=m)
                        cnt = cnt + plsc.all_reduce_population_count(m)[0]

                    @pl.when(cnt >= K)
                    def _():
                        process(g, jnp.int32(K))
                        for uu in range(4):
                            cid_b[pl.ds(uu * 16, 16)] = (
                                cid_b[pl.ds(K + uu * 16, 16)])
                            cdst_b[pl.ds(uu * 16, 16)] = (
                                cdst_b[pl.ds(K + uu * 16, 16)])

                    return jnp.where(cnt >= K, cnt - K, cnt)

                return lax.fori_loop(0, NVB // 4, vec4_body, cnt)

            cnt = lax.fori_loop(0, NBLK, blk_body, jnp.int32(0))

            @pl.when(cnt > 0)
            def _():
                process(g, cnt)

            pltpu.sync_copy(
                acc_v, outg_h.at[pl.ds(g * NP * DH + rowbase * DH, CR * DH)])
            return 0

        lax.fori_loop(0, SW, sweep_body, 0)
        return 0

    lax.fori_loop(0, NG, graph_body, 0)


def kernel(x, fg_edge_index, fg_pre_w, bg_edge_index, bg_pre_w,
           gg_edge_index, gg_pre_w, W_proj, att_l, att_r):
    f32 = jnp.float32
    i32 = jnp.int32

    A = jnp.concatenate([att_l.reshape(NG * HEADS, HID).T,
                         att_r.reshape(NG * HEADS, HID).T], axis=1)
    rb = 2000
    proj, S = pl.pallas_call(
        _dense_body,
        grid=(N // rb,),
        in_specs=[pl.BlockSpec((rb, FEAT), lambda i: (i, 0)),
                  pl.BlockSpec((FEAT, HID), lambda i: (0, 0)),
                  pl.BlockSpec((HID, 2 * NG * HEADS), lambda i: (0, 0))],
        out_specs=[pl.BlockSpec((rb, HID), lambda i: (i, 0)),
                   pl.BlockSpec((rb, 2 * NG * HEADS), lambda i: (i, 0))],
        out_shape=[jax.ShapeDtypeStruct((N, HID), f32),
                   jax.ShapeDtypeStruct((N, 2 * NG * HEADS), f32)],
    )(x, W_proj, A)

    SLm = jnp.transpose(S[:, :NG * HEADS].reshape(N, NG, HEADS), (1, 0, 2))
    SRm = jnp.transpose(S[:, NG * HEADS:].reshape(N, NG, HEADS), (1, 0, 2))
    SL = jnp.pad(SLm, ((0, 0), (0, NP - N), (0, 0))).reshape(NG * NP * HEADS)
    SR = jnp.pad(SRm, ((0, 0), (0, NP - N), (0, 0))).reshape(NG * NP * HEADS)
    srcs = jnp.concatenate([fg_edge_index[0], bg_edge_index[0],
                            gg_edge_index[0]]).astype(i32)
    dsts = jnp.concatenate([fg_edge_index[1], bg_edge_index[1],
                            gg_edge_index[1]]).astype(i32)
    prew = jnp.concatenate([fg_pre_w, bg_pre_w, gg_pre_w]).astype(f32)
    zden = jnp.zeros((HEADS * TSL,), f32)
    zacc = jnp.zeros((CR * DH,), f32)

    mesh = plsc.VectorSubcoreMesh(core_axis_name="c", subcore_axis_name="s")
    cparams = pltpu.CompilerParams(needs_layout_passes=False)

    wn0, wn1, wn2, wn3, denp = pl.kernel(
        _pass_a,
        out_type=(jax.ShapeDtypeStruct((NG * E,), f32),
                  jax.ShapeDtypeStruct((NG * E,), f32),
                  jax.ShapeDtypeStruct((NG * E,), f32),
                  jax.ShapeDtypeStruct((NG * E,), f32),
                  jax.ShapeDtypeStruct((2 * NG * HEADS * NP,), f32)),
        mesh=mesh,
        compiler_params=cparams,
        scratch_types=[
            pltpu.VMEM((NP * HEADS,), f32),
            pltpu.VMEM((NP * HEADS,), f32),
            pltpu.VMEM((EB,), i32),
            pltpu.VMEM((EB,), i32),
            pltpu.VMEM((EB,), f32),
            [pltpu.VMEM((EB,), f32) for _ in range(HEADS)],
            [pltpu.VMEM((EB,), f32) for _ in range(HEADS)],
            [pltpu.VMEM((EB,), i32) for _ in range(HEADS)],
            pltpu.VMEM_SHARED((HEADS * NP,), f32),
            pltpu.SemaphoreType.DMA,
        ],
    )(SL, SR, srcs, dsts, prew, zden)

    rden = pl.pallas_call(
        _rden_body,
        out_shape=jax.ShapeDtypeStruct((NG * HEADS * NP // 128, 128), f32),
    )(denp.reshape(2, NG * HEADS * NP // 128, 128)).reshape(NG * HEADS * NP)

    outg = pl.kernel(
        _pass_b,
        out_type=jax.ShapeDtypeStruct((NG * NP * DH,), f32),
        mesh=mesh,
        compiler_params=cparams,
        scratch_types=[
            pltpu.VMEM((SB,), i32),
            pltpu.VMEM((K + 64,), i32),
            pltpu.VMEM((K + 64,), i32),
            pltpu.VMEM((K,), i32),
            pltpu.VMEM((K,), i32),
            pltpu.VMEM((K,), i32),
            [pltpu.VMEM((K,), f32) for _ in range(HEADS)],
            [pltpu.VMEM((K,), f32) for _ in range(HEADS)],
            pltpu.VMEM((HEADS * CR,), f32),
            pltpu.VMEM((K, HID), f32),
            pltpu.VMEM((CR * DH,), f32),
            pltpu.SemaphoreType.DMA,
            pltpu.SemaphoreType.DMA,
        ],
    )(proj, srcs, dsts, wn0, wn1, wn2, wn3, rden, zacc)

    out = jnp.concatenate(
        [proj] + [outg.reshape(NG, NP, DH)[g, :N] for g in range(NG)],
        axis=-1)
    return out[None]
